# Initial kernel scaffold; baseline (speedup 1.0000x reference)
import jax
import jax.numpy as jnp
from jax.experimental import pallas as pl


def kernel(x, edge_index, e_emb, batch_index, W_Q, b_Q, W_K, b_K, W_V, b_V, W_Ew, b_Ew, W_Ev, b_Ev, W_A):
    raise NotImplementedError("write your pallas kernel here")


# speedup vs baseline: 20.4625x; 20.4625x over previous
import functools

import jax
import jax.numpy as jnp
from jax import lax
from jax.experimental import pallas as pl
from jax.experimental.pallas import tpu as pltpu
from jax.experimental.pallas import tpu_sc as plsc

N = 10000
D = 128
DE = 16
E = 320000
H = 4
DH = D // H
INV_S = 1.0 / (DH ** 0.5)

NC = 2
NS = 16
NW = NC * NS
EPW = E // NW
NP = 10240
COLS = NP // NS
CH = 80
EPT = E // NS
NCH = EPT // CH
DH2 = D // NC
AW = DH2 + DE
NRED = 8
CR = NP // NRED
SUB = CR // NS
NHALF = NP // 2
ACC_R = NHALF + 8
RPT = NHALF // NS
NB = 2
NG = NCH // NB

F32 = jnp.float32


def _prep_nodes_body(x_ref, wq_ref, wk_ref, wv_ref, bq_ref, bk_ref, bv_ref,
                     wa_ref, qa_ref, ka_ref, v_ref):
    wa = wa_ref[...]
    wqa = jnp.dot(wq_ref[...], wa, preferred_element_type=F32)
    wka = jnp.dot(wk_ref[...], wa, preferred_element_type=F32)
    bqa = jnp.dot(bq_ref[...], wa, preferred_element_type=F32)
    bka = jnp.dot(bk_ref[...], wa, preferred_element_type=F32)
    x = x_ref[...]
    qa_ref[...] = jnp.dot(x, wqa, preferred_element_type=F32) + bqa
    ka_ref[...] = jnp.dot(x, wka, preferred_element_type=F32) + bka
    v_ref[...] = jnp.dot(x, wv_ref[...], preferred_element_type=F32) + bv_ref[...]


def _prep_nodes(x_p, W_Q, W_K, W_V, b_Q, b_K, b_V, W_A):
    return pl.pallas_call(
        _prep_nodes_body,
        out_shape=[
            jax.ShapeDtypeStruct((NP, 1), F32),
            jax.ShapeDtypeStruct((NP, 1), F32),
            jax.ShapeDtypeStruct((NP, D), F32),
        ],
    )(x_p, W_Q, W_K, W_V, b_Q.reshape(1, D), b_K.reshape(1, D),
      b_V.reshape(1, D), W_A)


def _prep_edges_body(emb_ref, wew_ref, bew_ref, wa_ref, ea_ref):
    wa = wa_ref[...]
    wea = jnp.dot(wew_ref[...], wa, preferred_element_type=F32)
    bea = jnp.dot(bew_ref[...], wa, preferred_element_type=F32)
    ea_ref[...] = jnp.dot(emb_ref[...], wea, preferred_element_type=F32) + bea


def _prep_edges(e_emb, W_Ew, b_Ew, W_A):
    blk = E // 16
    return pl.pallas_call(
        _prep_edges_body,
        grid=(16,),
        in_specs=[
            pl.BlockSpec((blk, DE), lambda i: (i, 0)),
            pl.BlockSpec((DE, D), lambda i: (0, 0)),
            pl.BlockSpec((1, D), lambda i: (0, 0)),
            pl.BlockSpec((D, 1), lambda i: (0, 0)),
        ],
        out_specs=pl.BlockSpec((blk, 1), lambda i: (i, 0)),
        out_shape=jax.ShapeDtypeStruct((E, 1), F32),
    )(e_emb, W_Ew, b_Ew.reshape(1, D), W_A)


def _sc_pass1_body(src_h, dst_h, ea_h, qa_h, ka_h,
                   p_h, asum_h,
                   qa_v, ka_v, src_v, dst_v, ea_v, p_v, asum_v, rbuf, osl_v,
                   shared):
    c = lax.axis_index("c")
    s = lax.axis_index("s")
    wid = c * NS + s
    base = wid * EPW

    pltpu.sync_copy(qa_h, qa_v)
    pltpu.sync_copy(ka_h, ka_v)
    pltpu.sync_copy(src_h.at[pl.ds(base, EPW)], src_v)
    pltpu.sync_copy(dst_h.at[pl.ds(base, EPW)], dst_v)
    pltpu.sync_copy(ea_h.at[pl.ds(base, EPW)], ea_v)

    zero16 = jnp.zeros((16,), F32)

    def zbody(i, carry):
        asum_v[pl.ds(i * 16, 16)] = zero16
        return carry
    lax.fori_loop(0, NP // 16, zbody, 0)

    def ebody(i, carry):
        sl = pl.ds(i * 16, 16)
        si = src_v[sl]
        di = dst_v[sl]
        qg = plsc.load_gather(qa_v, [si])
        kg = plsc.load_gather(ka_v, [di])
        pv = jnp.exp((qg + kg + ea_v[sl]) * INV_S)
        p_v[sl] = pv
        plsc.addupdate_scatter(asum_v, [di], pv)
        return carry
    lax.fori_loop(0, EPW // 16, ebody, 0)

    pltpu.sync_copy(p_v, p_h.at[pl.ds(base, EPW)])

    for r in range(NRED):
        pltpu.sync_copy(asum_v.at[pl.ds(r * CR, CR)], shared.at[s])
        plsc.subcore_barrier()
        for j in range(NS):
            pltpu.sync_copy(shared.at[j, pl.ds(s * SUB, SUB)], rbuf.at[j])

        def cbody(t, carry):
            sl = pl.ds(t * 16, 16)
            acc = rbuf[0, sl]
            for j in range(1, NS):
                acc = acc + rbuf[j, sl]
            osl_v[sl] = acc
            return carry
        lax.fori_loop(0, SUB // 16, cbody, 0)
        pltpu.sync_copy(osl_v, asum_h.at[c, pl.ds(r * CR + s * SUB, SUB)])
        plsc.subcore_barrier()


def _sc_pass1(src, dst, ea, qa, ka):
    mesh = plsc.VectorSubcoreMesh(core_axis_name="c", subcore_axis_name="s",
                                  num_cores=NC, num_subcores=NS)
    return pl.kernel(
        _sc_pass1_body,
        out_type=[
            jax.ShapeDtypeStruct((E,), F32),
            jax.ShapeDtypeStruct((NC, NP), F32),
        ],
        mesh=mesh,
        scratch_types=[
            pltpu.VMEM((NP,), F32),
            pltpu.VMEM((NP,), F32),
            pltpu.VMEM((EPW,), jnp.int32),
            pltpu.VMEM((EPW,), jnp.int32),
            pltpu.VMEM((EPW,), F32),
            pltpu.VMEM((EPW,), F32),
            pltpu.VMEM((NP,), F32),
            pltpu.VMEM((NS, SUB), F32),
            pltpu.VMEM((SUB,), F32),
            pltpu.VMEM_SHARED((NS, CR), F32),
        ],
        compiler_params=pltpu.CompilerParams(needs_layout_passes=False,
                                             use_tc_tiling_on_sc=False),
    )(src, dst, ea, qa, ka)


def _sc_pass2_body(src_h, dst_h, p_h, asum2_h, emb_h, vr_h,
                   acc_h, t_h,
                   asum_v, src_v, dst_v, alpha_v,
                   g64, g_v, srcc, idxc, emb_c, sg0, sg1, ss0, ss1,
                   acc):
    sem_g = [sg0, sg1]
    sem_s = [ss0, ss1]
    c = lax.axis_index("c")
    s = lax.axis_index("s")
    base = s * EPT

    pltpu.sync_copy(asum2_h.at[0], asum_v)
    pltpu.sync_copy(asum2_h.at[1], alpha_v.at[pl.ds(0, NP)])

    def abody(i, carry):
        sl = pl.ds(i * 16, 16)
        asum_v[sl] = asum_v[sl] + alpha_v[sl]
        return carry
    lax.fori_loop(0, NP // 16, abody, 0)

    pltpu.sync_copy(src_h.at[pl.ds(base, EPT)], src_v)
    pltpu.sync_copy(dst_h.at[pl.ds(base, EPT)], dst_v)
    pltpu.sync_copy(p_h.at[pl.ds(base, EPT)], alpha_v)

    zero16 = jnp.zeros((16,), F32)

    def p1(i, carry):
        sl = pl.ds(i * 16, 16)
        di = dst_v[sl]
        ag = plsc.load_gather(asum_v, [di])
        alpha_v[sl] = alpha_v[sl] / (ag + 1e-9)
        return carry
    lax.fori_loop(0, EPT // 16, p1, 0)

    @pl.when(c == 1)
    def _t_part():
        def ztbody(i, carry):
            asum_v[pl.ds(i * 16, 16)] = zero16
            return carry
        lax.fori_loop(0, NP // 16, ztbody, 0)

        def tbody(i, carry):
            sl = pl.ds(i * 16, 16)
            plsc.addupdate_scatter(asum_v, [src_v[sl]], alpha_v[sl])
            return carry
        lax.fori_loop(0, EPT // 16, tbody, 0)
        pltpu.sync_copy(asum_v, t_h.at[s])

    def stage_idx(b, cix):
        e0 = cix * CH
        for t in range(CH // 16):
            sl = pl.ds(t * 16, 16)
            sle = pl.ds(e0 + t * 16, 16)
            idxc[b, sl] = dst_v[sle] * 2 + c
    def start_gather(b):
        pltpu.make_async_copy(vr_h.at[idxc.at[b]], g64.at[b],
                              sem_g[b]).start()

    def round_body(r, rcarry):
        def zg(i, carry):
            rr = i // (AW // 16)
            k = i - rr * (AW // 16)
            g_v[0, rr, pl.ds(k * 16, 16)] = zero16
            return carry
        lax.fori_loop(0, CH * (AW // 16), zg, 0)

        def zo(i, carry):
            pltpu.sync_copy(g_v.at[0], acc.at[pl.ds(s * RPT + i * CH, CH), :])
            return carry
        lax.fori_loop(0, RPT // CH, zo, 0)
        plsc.subcore_barrier()

        for b in range(NB):
            stage_idx(b, b)
            start_gather(b)

        def grp(g, carry):
            for b in range(NB):
                cix = g * NB + b
                e0 = cix * CH
                pltpu.make_async_copy(vr_h.at[idxc.at[b]], g64.at[b],
                                      sem_g[b]).wait()
                pltpu.sync_copy(emb_h.at[pl.ds(base + e0, CH)], emb_c)

                @pl.when(cix >= NB)
                def _wait_prev_scatter():
                    pltpu.make_async_copy(g_v.at[b], acc.at[srcc.at[b]],
                                          sem_s[b]).wait()

                for t in range(CH // 16):
                    sl = pl.ds(t * 16, 16)
                    sle = pl.ds(e0 + t * 16, 16)
                    sv = src_v[sle] - (r * NHALF)
                    ok = (sv >= 0) & (sv < NHALF)
                    srcc[b, sl] = jnp.where(ok, sv, NHALF)

                @functools.partial(plsc.parallel_loop, 0, CH // 16)
                def rbody(jj):
                    av16 = alpha_v[pl.ds(e0 + jj * 16, 16)]
                    for j in range(16):
                        av = av16.at[jnp.full((16,), j, jnp.int32)].get(
                            mode="promise_in_bounds")
                        row = jj * 16 + j
                        for k in range(DH2 // 16):
                            sl = pl.ds(k * 16, 16)
                            g_v[b, row, sl] = g64[b, row, sl] * av
                        g_v[b, row, pl.ds(DH2, 16)] = emb_c[row, :] * av
                pltpu.async_copy(g_v.at[b], acc.at[srcc.at[b]],
                                 sem_s[b], add=True)

                @pl.when(cix + NB < NCH)
                def _prefetch():
                    stage_idx(b, cix + NB)
                    start_gather(b)
            return carry
        lax.fori_loop(0, NG, grp, 0)
        for b in range(NB):
            pltpu.make_async_copy(g_v.at[b], acc.at[srcc.at[b]],
                                  sem_s[b]).wait()
        plsc.subcore_barrier()

        pltpu.sync_copy(acc.at[pl.ds(s * RPT, RPT), :],
                        acc_h.at[c, pl.ds(r * NHALF + s * RPT, RPT), :])
        plsc.subcore_barrier()
        return rcarry
    lax.fori_loop(0, 2, round_body, 0)


def _sc_pass2(src, dst, p, asum_parts, e_emb, V_r):
    mesh = plsc.VectorSubcoreMesh(core_axis_name="c", subcore_axis_name="s",
                                  num_cores=NC, num_subcores=NS)
    return pl.kernel(
        _sc_pass2_body,
        out_type=[
            jax.ShapeDtypeStruct((NC, NP, AW), F32),
            jax.ShapeDtypeStruct((NS, NP), F32),
        ],
        mesh=mesh,
        scratch_types=[
            pltpu.VMEM((NP,), F32),
            pltpu.VMEM((EPT,), jnp.int32),
            pltpu.VMEM((EPT,), jnp.int32),
            pltpu.VMEM((EPT,), F32),
            pltpu.VMEM((NB, CH, DH2), F32),
            pltpu.VMEM((NB, CH, AW), F32),
            pltpu.VMEM((NB, CH), jnp.int32),
            pltpu.VMEM((NB, CH), jnp.int32),
            pltpu.VMEM((CH, DE), F32),
            pltpu.SemaphoreType.DMA,
            pltpu.SemaphoreType.DMA,
            pltpu.SemaphoreType.DMA,
            pltpu.SemaphoreType.DMA,
            pltpu.VMEM_SHARED((ACC_R, AW), F32),
        ],
        compiler_params=pltpu.CompilerParams(needs_layout_passes=False,
                                             use_tc_tiling_on_sc=False),
    )(src, dst, p, asum_parts, e_emb, V_r)


def _combine_body(acc_ref, t_ref, wev_ref, bev_ref, out_ref):
    o = jnp.concatenate([acc_ref[0, :, :DH2], acc_ref[1, :, :DH2]], axis=1)
    sm = acc_ref[0, :, DH2:]
    tt = jnp.sum(t_ref[...], axis=0)
    o = o + jnp.dot(sm, wev_ref[...], preferred_element_type=F32)
    out_ref[...] = o + tt[:, None] * bev_ref[...]


def _combine(acc_parts, t_parts, W_Ev, b_Ev):
    blk = NP // 8
    return pl.pallas_call(
        _combine_body,
        grid=(8,),
        in_specs=[
            pl.BlockSpec((NC, blk, AW), lambda i: (0, i, 0)),
            pl.BlockSpec((NS, blk), lambda i: (0, i)),
            pl.BlockSpec((DE, D), lambda i: (0, 0)),
            pl.BlockSpec((1, D), lambda i: (0, 0)),
        ],
        out_specs=pl.BlockSpec((blk, D), lambda i: (i, 0)),
        out_shape=jax.ShapeDtypeStruct((NP, D), F32),
    )(acc_parts, t_parts, W_Ev, b_Ev.reshape(1, D))


@jax.jit
def _kernel_impl(x, edge_index, e_emb, W_Q, b_Q, W_K, b_K, W_V, b_V,
                 W_Ew, b_Ew, W_Ev, b_Ev, W_A):
    x2 = x.reshape(N, D)
    x_p = jnp.pad(x2, ((0, NP - N), (0, 0)))
    src = edge_index[0]
    dst = edge_index[1]

    qa, ka, V = _prep_nodes(x_p, W_Q, W_K, W_V, b_Q, b_K, b_V, W_A)
    ea = _prep_edges(e_emb, W_Ew, b_Ew, W_A)

    p, asum_parts = _sc_pass1(src, dst, ea.reshape(E), qa.reshape(NP),
                              ka.reshape(NP))
    V_r = V.reshape(NP, NC, DH2).reshape(NC * NP, DH2)
    acc_parts, t_parts = _sc_pass2(src, dst, p, asum_parts, e_emb, V_r)
    out_np = _combine(acc_parts, t_parts, W_Ev, b_Ev)
    return out_np[:N].reshape(1, N, D)


def kernel(x, edge_index, e_emb, batch_index, W_Q, b_Q, W_K, b_K, W_V, b_V,
           W_Ew, b_Ew, W_Ev, b_Ev, W_A):
    del batch_index
    return _kernel_impl(x, edge_index, e_emb, W_Q, b_Q, W_K, b_K, W_V, b_V,
                        W_Ew, b_Ew, W_Ev, b_Ev, W_A)
